# baseline (speedup 1.0000x reference)
import functools

import jax
import jax.numpy as jnp
from jax.experimental import pallas as pl


def _one_batch(q, p, L):
    M = jax.lax.dot_general(
        p, q, (((1,), (1,)), ((), ())), preferred_element_type=jnp.float32
    )

    rowmax = jnp.max(M, axis=1, keepdims=True)
    colmax = jnp.max(M, axis=0, keepdims=True)
    ymax = jnp.max(
        jnp.where(M == rowmax, M, -jnp.inf), axis=0, keepdims=True
    )
    validf = (ymax == colmax).astype(jnp.float32)
    count = jnp.sum(validf)
    masked_sum = jnp.sum(validf * colmax)
    masked_mean = masked_sum / jnp.maximum(count, 1.0)

    fallback = jnp.sum(q * p) / jnp.float32(L)
    return jnp.where(count > 1.0, masked_mean, fallback)


def _loss_kernel(q_ref, p_ref, out_ref, *, L, NB):
    step = pl.program_id(0)
    for j in range(NB):
        r = _one_batch(q_ref[j], p_ref[j], L)
        out_ref[pl.ds(step * NB + j, 1), :] = r.reshape(1, 1)


def kernel(feats1, feats2):
    B, H, W, C = feats2.shape
    L = H * W
    NB = 8
    q = feats1.reshape(B, L, C)
    p = feats2.reshape(B, L, C)

    out = pl.pallas_call(
        functools.partial(_loss_kernel, L=L, NB=NB),
        grid=(B // NB,),
        in_specs=[
            pl.BlockSpec((NB, L, C), lambda b: (b, 0, 0)),
            pl.BlockSpec((NB, L, C), lambda b: (b, 0, 0)),
        ],
        out_specs=pl.BlockSpec((B, 1), lambda b: (0, 0)),
        out_shape=jax.ShapeDtypeStruct((B, 1), jnp.float32),
    )(q, p)
    return out[:, 0]


# speedup vs baseline: 1.0491x; 1.0491x over previous
import functools

import jax
import jax.numpy as jnp
from jax.experimental import pallas as pl


def _one_batch(q, p, L):
    M = jax.lax.dot_general(
        p, q, (((1,), (1,)), ((), ())), preferred_element_type=jnp.float32
    )

    rowmax = jnp.max(M, axis=1, keepdims=True)
    colmax = jnp.max(M, axis=0, keepdims=True)
    ymax = jnp.max(
        jnp.where(M == rowmax, M, -jnp.inf), axis=0, keepdims=True
    )
    validf = (ymax == colmax).astype(jnp.float32)
    count = jnp.sum(validf)
    masked_sum = jnp.sum(validf * colmax)
    masked_mean = masked_sum / jnp.maximum(count, 1.0)

    fallback = jnp.sum(q * p) / jnp.float32(L)
    return jnp.where(count > 1.0, masked_mean, fallback)


def _loss_kernel(q_ref, p_ref, out_ref, *, L, NB):
    step = pl.program_id(0)
    for j in range(NB):
        r = _one_batch(q_ref[j], p_ref[j], L)
        out_ref[pl.ds(step * NB + j, 1), :] = r.reshape(1, 1)


def kernel(feats1, feats2):
    B, H, W, C = feats2.shape
    L = H * W
    NB = 4
    q = feats1.reshape(B, L, C)
    p = feats2.reshape(B, L, C)

    out = pl.pallas_call(
        functools.partial(_loss_kernel, L=L, NB=NB),
        grid=(B // NB,),
        in_specs=[
            pl.BlockSpec((NB, L, C), lambda b: (b, 0, 0)),
            pl.BlockSpec((NB, L, C), lambda b: (b, 0, 0)),
        ],
        out_specs=pl.BlockSpec((B, 1), lambda b: (0, 0)),
        out_shape=jax.ShapeDtypeStruct((B, 1), jnp.float32),
    )(q, p)
    return out[:, 0]


# speedup vs baseline: 1.0597x; 1.0100x over previous
import functools

import jax
import jax.numpy as jnp
from jax.experimental import pallas as pl


def _one_batch(q, p, M, L):
    rowmax = jnp.max(M, axis=1, keepdims=True)
    colmax = jnp.max(M, axis=0, keepdims=True)
    ymax = jnp.max(
        jnp.where(M == rowmax, M, -jnp.inf), axis=0, keepdims=True
    )
    validf = (ymax == colmax).astype(jnp.float32)
    count = jnp.sum(validf)
    masked_sum = jnp.sum(validf * colmax)
    masked_mean = masked_sum / jnp.maximum(count, 1.0)

    fallback = jnp.sum(q * p) / jnp.float32(L)
    return jnp.where(count > 1.0, masked_mean, fallback)


def _loss_kernel(q_ref, p_ref, out_ref, *, L, NB):
    step = pl.program_id(0)
    for j in range(NB):
        M = jax.lax.dot_general(
            p_ref[j], q_ref[j], (((1,), (1,)), ((), ())),
            preferred_element_type=jnp.float32,
        )
        r = _one_batch(q_ref[j], p_ref[j], M, L)
        out_ref[pl.ds(step * NB + j, 1), :] = r.reshape(1, 1)


def kernel(feats1, feats2):
    B, H, W, C = feats2.shape
    L = H * W
    NB = 4
    q = feats1.reshape(B, L, C)
    p = feats2.reshape(B, L, C)

    out = pl.pallas_call(
        functools.partial(_loss_kernel, L=L, NB=NB),
        grid=(B // NB,),
        in_specs=[
            pl.BlockSpec((NB, L, C), lambda b: (b, 0, 0)),
            pl.BlockSpec((NB, L, C), lambda b: (b, 0, 0)),
        ],
        out_specs=pl.BlockSpec((B, 1), lambda b: (0, 0)),
        out_shape=jax.ShapeDtypeStruct((B, 1), jnp.float32),
    )(q, p)
    return out[:, 0]
